# baseline (speedup 1.0000x reference)
import jax
import jax.numpy as jnp
from jax import lax
from jax.experimental import pallas as pl
from jax.experimental.pallas import tpu as pltpu
from jax.experimental.pallas import tpu_sc as plsc

N_CORES = 2
N_SUBCORES = 16
NW = N_CORES * N_SUBCORES
L = 16

BATCH = 16384
D = 64
BLK = 2 * D
HALF = 50048
NBLK = HALF // 128
BPW = BATCH // NW
CHUNK = 128
N_CHUNKS = BPW // CHUNK
GROUPS = CHUNK // L


PACK_COLS = 2176
PACK_GRID = HALF // PACK_COLS


def _tc_pack_body(t1_ref, t2_ref, out_ref):
    ey = (lax.broadcasted_iota(jnp.int32, (D, D), 0) ==
          lax.broadcasted_iota(jnp.int32, (D, D), 1)).astype(jnp.float32)
    dn = (((0,), (0,)), ((), ()))
    out_ref[:, 0:D] = lax.dot_general(t1_ref[...], ey, dn,
                                      precision=lax.Precision.HIGHEST,
                                      preferred_element_type=jnp.float32)
    out_ref[:, D:BLK] = lax.dot_general(t2_ref[...], ey, dn,
                                        precision=lax.Precision.HIGHEST,
                                        preferred_element_type=jnp.float32)


def _pack_table(table_t):
    return pl.pallas_call(
        _tc_pack_body,
        out_shape=jax.ShapeDtypeStruct((HALF, BLK), jnp.float32),
        grid=(PACK_GRID,),
        in_specs=[
            pl.BlockSpec((D, PACK_COLS), lambda j: (0, j)),
            pl.BlockSpec((D, PACK_COLS), lambda j: (0, j + PACK_GRID)),
        ],
        out_specs=pl.BlockSpec((PACK_COLS, BLK), lambda j: (j, 0)),
    )(table_t, table_t)


def _permute(v, idx):
    dn = lax.GatherDimensionNumbers(offset_dims=(), collapsed_slice_dims=(0,),
                                    start_index_map=(0,))
    return lax.gather(v, idx[:, None], dn, (1,),
                      mode=lax.GatherScatterMode.PROMISE_IN_BOUNDS)


def _sc_body(nid_hbm, nnid_hbm, table_hbm, bias_hbm, out_hbm,
             idx_a, idx_b, blk_a, blk_b, rid_b, rows_a, rows_b, bias_v, out_v,
             sem0, sem1):
    wid = lax.axis_index("s") * N_CORES + lax.axis_index("c")
    base = wid * BPW

    pltpu.sync_copy(nid_hbm.at[pl.ds(base, BPW)], idx_a)
    pltpu.sync_copy(nnid_hbm.at[pl.ds(base, BPW)], idx_b)

    for i in range(BPW // L):
        c, w = divmod(i, CHUNK // L)
        va = idx_a[pl.ds(i * L, L)]
        vb = idx_b[pl.ds(i * L, L)]
        blk_a[c, pl.ds(w * L, L)] = jnp.where(va < HALF, va, va - HALF)
        blk_b[c, pl.ds(w * L, L)] = jnp.where(vb < HALF, vb, vb - HALF)
        rid_b[c, pl.ds(w * L, L)] = vb

    sems = [sem0, sem1]
    lanes = lax.iota(jnp.int32, L)

    def fire(c):
        buf = c % 2
        s = sems[buf]
        return [
            pltpu.async_copy(table_hbm.at[blk_a.at[c]], rows_a.at[buf], s),
            pltpu.async_copy(table_hbm.at[blk_b.at[c]], rows_b.at[buf], s),
            pltpu.async_copy(bias_hbm.at[rid_b.at[c]],
                             bias_v.at[pl.ds(c * CHUNK, CHUNK)], s),
        ]

    pending = {0: fire(0)}
    for c in range(N_CHUNKS):
        if c + 1 < N_CHUNKS:
            pending[c + 1] = fire(c + 1)
        for cp in pending.pop(c):
            cp.wait()
        buf = c % 2

        def group_body(g, carry, buf=buf, c=c):
            rbase = g * L
            abs_base = c * CHUNK + rbase
            off_a = jnp.where(idx_a[pl.ds(abs_base, L)] < HALF, 0, D)
            off_b = jnp.where(idx_b[pl.ds(abs_base, L)] < HALF, 0, D)
            vs = []
            for r in range(L):
                oa = off_a[r]
                ob = off_b[r]
                row = rbase + r
                acc = (rows_a[buf, row, pl.ds(oa, L)] *
                       rows_b[buf, row, pl.ds(ob, L)])
                for k in range(1, D // L):
                    acc = acc + (rows_a[buf, row, pl.ds(oa + k * L, L)] *
                                 rows_b[buf, row, pl.ds(ob + k * L, L)])
                vs.append(acc)
            s = L // 2
            while s >= 1:
                mask = (lanes & s) == 0
                pidx = lanes ^ s
                nxt = []
                for i in range(s):
                    a = vs[i] + _permute(vs[i], pidx)
                    b = vs[i + s] + _permute(vs[i + s], pidx)
                    nxt.append(jnp.where(mask, a, b))
                vs = nxt
                s //= 2
            out_v[pl.ds(abs_base, L)] = vs[0] + bias_v[pl.ds(abs_base, L)]
            return carry

        lax.fori_loop(0, GROUPS, group_body, 0)

    pltpu.sync_copy(out_v, out_hbm.at[pl.ds(base, BPW)])


def _sc_gather_dot(nid, nnid, table_p, bias):
    mesh = plsc.VectorSubcoreMesh(core_axis_name="c", subcore_axis_name="s")
    return pl.kernel(
        _sc_body,
        out_type=jax.ShapeDtypeStruct((BATCH,), jnp.float32),
        mesh=mesh,
        scratch_types=[
            pltpu.VMEM((BPW,), jnp.int32),
            pltpu.VMEM((BPW,), jnp.int32),
            pltpu.VMEM((N_CHUNKS, CHUNK), jnp.int32),
            pltpu.VMEM((N_CHUNKS, CHUNK), jnp.int32),
            pltpu.VMEM((N_CHUNKS, CHUNK), jnp.int32),
            pltpu.VMEM((2, CHUNK, BLK), jnp.float32),
            pltpu.VMEM((2, CHUNK, BLK), jnp.float32),
            pltpu.VMEM((BPW,), jnp.float32),
            pltpu.VMEM((BPW,), jnp.float32),
            pltpu.SemaphoreType.DMA,
            pltpu.SemaphoreType.DMA,
        ],
        compiler_params=pltpu.CompilerParams(use_tc_tiling_on_sc=True),
    )(nid, nnid, table_p, bias)


@jax.jit
def _run(node_id, node_neighbor_id, embedding_matrix, bias):
    nid = node_id.astype(jnp.int32)
    nnid = node_neighbor_id.astype(jnp.int32)
    table_p = _pack_table(embedding_matrix.T)
    return _sc_gather_dot(nid, nnid, table_p, bias)


def kernel(node_id, node_neighbor_id, embedding_matrix, bias):
    return _run(node_id, node_neighbor_id, embedding_matrix, bias)


# speedup vs baseline: 1.3516x; 1.3516x over previous
import jax
import jax.numpy as jnp
from jax import lax
from jax.experimental import pallas as pl
from jax.experimental.pallas import tpu as pltpu
from jax.experimental.pallas import tpu_sc as plsc

N_CORES = 2
N_SUBCORES = 16
NW = N_CORES * N_SUBCORES
L = 16

BATCH = 16384
D = 64
BLK = 2 * D
HALF = 50048
NBLK = HALF // 128
BPW = BATCH // NW
CHUNK = 128
N_CHUNKS = BPW // CHUNK
GROUPS = CHUNK // L


PACK_COLS = 2176
PACK_GRID = HALF // PACK_COLS


def _tc_pack_body(t1_ref, t2_ref, out_ref):
    out_ref[:, 0:D] = t1_ref[...].T
    out_ref[:, D:BLK] = t2_ref[...].T


def _pack_table(table_t):
    return pl.pallas_call(
        _tc_pack_body,
        out_shape=jax.ShapeDtypeStruct((HALF, BLK), jnp.float32),
        grid=(PACK_GRID,),
        in_specs=[
            pl.BlockSpec((D, PACK_COLS), lambda j: (0, j)),
            pl.BlockSpec((D, PACK_COLS), lambda j: (0, j + PACK_GRID)),
        ],
        out_specs=pl.BlockSpec((PACK_COLS, BLK), lambda j: (j, 0)),
    )(table_t, table_t)


def _permute(v, idx):
    dn = lax.GatherDimensionNumbers(offset_dims=(), collapsed_slice_dims=(0,),
                                    start_index_map=(0,))
    return lax.gather(v, idx[:, None], dn, (1,),
                      mode=lax.GatherScatterMode.PROMISE_IN_BOUNDS)


def _sc_body(nid_hbm, nnid_hbm, table_hbm, bias_hbm, out_hbm,
             idx_a, idx_b, blk_a, blk_b, rid_b, rows_a, rows_b, bias_v, out_v,
             sem0, sem1):
    wid = lax.axis_index("s") * N_CORES + lax.axis_index("c")
    base = wid * BPW

    pltpu.sync_copy(nid_hbm.at[pl.ds(base, BPW)], idx_a)
    pltpu.sync_copy(nnid_hbm.at[pl.ds(base, BPW)], idx_b)

    for i in range(BPW // L):
        c, w = divmod(i, CHUNK // L)
        va = idx_a[pl.ds(i * L, L)]
        vb = idx_b[pl.ds(i * L, L)]
        blk_a[c, pl.ds(w * L, L)] = jnp.where(va < HALF, va, va - HALF)
        blk_b[c, pl.ds(w * L, L)] = jnp.where(vb < HALF, vb, vb - HALF)
        rid_b[c, pl.ds(w * L, L)] = vb

    sems = [sem0, sem1]
    lanes = lax.iota(jnp.int32, L)

    def fire(c):
        buf = c % 2
        s = sems[buf]
        return [
            pltpu.async_copy(table_hbm.at[blk_a.at[c]], rows_a.at[buf], s),
            pltpu.async_copy(table_hbm.at[blk_b.at[c]], rows_b.at[buf], s),
            pltpu.async_copy(bias_hbm.at[rid_b.at[c]],
                             bias_v.at[pl.ds(c * CHUNK, CHUNK)], s),
        ]

    pending = {0: fire(0)}
    for c in range(N_CHUNKS):
        if c + 1 < N_CHUNKS:
            pending[c + 1] = fire(c + 1)
        for cp in pending.pop(c):
            cp.wait()
        buf = c % 2

        def group_body(g, carry, buf=buf, c=c):
            rbase = g * L
            abs_base = c * CHUNK + rbase
            off_a = jnp.where(idx_a[pl.ds(abs_base, L)] < HALF, 0, D)
            off_b = jnp.where(idx_b[pl.ds(abs_base, L)] < HALF, 0, D)
            vs = []
            for r in range(L):
                oa = off_a[r]
                ob = off_b[r]
                row = rbase + r
                acc = (rows_a[buf, row, pl.ds(oa, L)] *
                       rows_b[buf, row, pl.ds(ob, L)])
                for k in range(1, D // L):
                    acc = acc + (rows_a[buf, row, pl.ds(oa + k * L, L)] *
                                 rows_b[buf, row, pl.ds(ob + k * L, L)])
                vs.append(acc)
            s = L // 2
            while s >= 1:
                mask = (lanes & s) == 0
                pidx = lanes ^ s
                nxt = []
                for i in range(s):
                    a = vs[i] + _permute(vs[i], pidx)
                    b = vs[i + s] + _permute(vs[i + s], pidx)
                    nxt.append(jnp.where(mask, a, b))
                vs = nxt
                s //= 2
            out_v[pl.ds(abs_base, L)] = vs[0] + bias_v[pl.ds(abs_base, L)]
            return carry

        lax.fori_loop(0, GROUPS, group_body, 0)

    pltpu.sync_copy(out_v, out_hbm.at[pl.ds(base, BPW)])


def _sc_gather_dot(nid, nnid, table_p, bias):
    mesh = plsc.VectorSubcoreMesh(core_axis_name="c", subcore_axis_name="s")
    return pl.kernel(
        _sc_body,
        out_type=jax.ShapeDtypeStruct((BATCH,), jnp.float32),
        mesh=mesh,
        scratch_types=[
            pltpu.VMEM((BPW,), jnp.int32),
            pltpu.VMEM((BPW,), jnp.int32),
            pltpu.VMEM((N_CHUNKS, CHUNK), jnp.int32),
            pltpu.VMEM((N_CHUNKS, CHUNK), jnp.int32),
            pltpu.VMEM((N_CHUNKS, CHUNK), jnp.int32),
            pltpu.VMEM((2, CHUNK, BLK), jnp.float32),
            pltpu.VMEM((2, CHUNK, BLK), jnp.float32),
            pltpu.VMEM((BPW,), jnp.float32),
            pltpu.VMEM((BPW,), jnp.float32),
            pltpu.SemaphoreType.DMA,
            pltpu.SemaphoreType.DMA,
        ],
        compiler_params=pltpu.CompilerParams(use_tc_tiling_on_sc=True),
    )(nid, nnid, table_p, bias)


@jax.jit
def _run(node_id, node_neighbor_id, embedding_matrix, bias):
    nid = node_id.astype(jnp.int32)
    nnid = node_neighbor_id.astype(jnp.int32)
    table_p = _pack_table(embedding_matrix.T)
    return _sc_gather_dot(nid, nnid, table_p, bias)


def kernel(node_id, node_neighbor_id, embedding_matrix, bias):
    return _run(node_id, node_neighbor_id, embedding_matrix, bias)


# speedup vs baseline: 1.4205x; 1.0510x over previous
import jax
import jax.numpy as jnp
from jax import lax
from jax.experimental import pallas as pl
from jax.experimental.pallas import tpu as pltpu
from jax.experimental.pallas import tpu_sc as plsc

N_CORES = 2
N_SUBCORES = 16
NW = N_CORES * N_SUBCORES
L = 16

BATCH = 16384
D = 64
BLK = 2 * D
HALF = 50048
NBLK = HALF // 128
BPW = BATCH // NW
CHUNK = 128
N_CHUNKS = BPW // CHUNK
GROUPS = CHUNK // L


PACK_COLS = 2944
PACK_GRID = HALF // PACK_COLS


def _tc_pack_body(t1_ref, t2_ref, out_ref):
    out_ref[:, 0:D] = t1_ref[...].T
    out_ref[:, D:BLK] = t2_ref[...].T


def _pack_table(table_t):
    return pl.pallas_call(
        _tc_pack_body,
        out_shape=jax.ShapeDtypeStruct((HALF, BLK), jnp.float32),
        grid=(PACK_GRID,),
        in_specs=[
            pl.BlockSpec((D, PACK_COLS), lambda j: (0, j)),
            pl.BlockSpec((D, PACK_COLS), lambda j: (0, j + PACK_GRID)),
        ],
        out_specs=pl.BlockSpec((PACK_COLS, BLK), lambda j: (j, 0)),
    )(table_t, table_t)


def _permute(v, idx):
    dn = lax.GatherDimensionNumbers(offset_dims=(), collapsed_slice_dims=(0,),
                                    start_index_map=(0,))
    return lax.gather(v, idx[:, None], dn, (1,),
                      mode=lax.GatherScatterMode.PROMISE_IN_BOUNDS)


def _sc_body(nid_hbm, nnid_hbm, table_hbm, bias_hbm, out_hbm,
             idx_a, idx_b, blk_a, blk_b, rid_b, rows_a, rows_b, bias_v, out_v,
             sem0, sem1):
    wid = lax.axis_index("s") * N_CORES + lax.axis_index("c")
    base = wid * BPW

    pltpu.sync_copy(nid_hbm.at[pl.ds(base, BPW)], idx_a)
    pltpu.sync_copy(nnid_hbm.at[pl.ds(base, BPW)], idx_b)

    for i in range(BPW // L):
        c, w = divmod(i, CHUNK // L)
        va = idx_a[pl.ds(i * L, L)]
        vb = idx_b[pl.ds(i * L, L)]
        blk_a[c, pl.ds(w * L, L)] = jnp.where(va < HALF, va, va - HALF)
        blk_b[c, pl.ds(w * L, L)] = jnp.where(vb < HALF, vb, vb - HALF)
        rid_b[c, pl.ds(w * L, L)] = vb

    sems = [sem0, sem1]
    lanes = lax.iota(jnp.int32, L)

    def fire(c):
        buf = c % 2
        s = sems[buf]
        return [
            pltpu.async_copy(table_hbm.at[blk_a.at[c]], rows_a.at[buf], s),
            pltpu.async_copy(table_hbm.at[blk_b.at[c]], rows_b.at[buf], s),
            pltpu.async_copy(bias_hbm.at[rid_b.at[c]],
                             bias_v.at[pl.ds(c * CHUNK, CHUNK)], s),
        ]

    pending = {0: fire(0)}
    for c in range(N_CHUNKS):
        if c + 1 < N_CHUNKS:
            pending[c + 1] = fire(c + 1)
        for cp in pending.pop(c):
            cp.wait()
        buf = c % 2

        def group_body(g, carry, buf=buf, c=c):
            rbase = g * L
            abs_base = c * CHUNK + rbase
            off_a = jnp.where(idx_a[pl.ds(abs_base, L)] < HALF, 0, D)
            off_b = jnp.where(idx_b[pl.ds(abs_base, L)] < HALF, 0, D)
            vs = []
            for r in range(L):
                oa = off_a[r]
                ob = off_b[r]
                row = rbase + r
                acc = (rows_a[buf, row, pl.ds(oa, L)] *
                       rows_b[buf, row, pl.ds(ob, L)])
                for k in range(1, D // L):
                    acc = acc + (rows_a[buf, row, pl.ds(oa + k * L, L)] *
                                 rows_b[buf, row, pl.ds(ob + k * L, L)])
                vs.append(acc)
            s = L // 2
            while s >= 1:
                mask = (lanes & s) == 0
                pidx = lanes ^ s
                nxt = []
                for i in range(s):
                    a = vs[i] + _permute(vs[i], pidx)
                    b = vs[i + s] + _permute(vs[i + s], pidx)
                    nxt.append(jnp.where(mask, a, b))
                vs = nxt
                s //= 2
            out_v[pl.ds(abs_base, L)] = vs[0] + bias_v[pl.ds(abs_base, L)]
            return carry

        lax.fori_loop(0, GROUPS, group_body, 0)

    pltpu.sync_copy(out_v, out_hbm.at[pl.ds(base, BPW)])


def _sc_gather_dot(nid, nnid, table_p, bias):
    mesh = plsc.VectorSubcoreMesh(core_axis_name="c", subcore_axis_name="s")
    return pl.kernel(
        _sc_body,
        out_type=jax.ShapeDtypeStruct((BATCH,), jnp.float32),
        mesh=mesh,
        scratch_types=[
            pltpu.VMEM((BPW,), jnp.int32),
            pltpu.VMEM((BPW,), jnp.int32),
            pltpu.VMEM((N_CHUNKS, CHUNK), jnp.int32),
            pltpu.VMEM((N_CHUNKS, CHUNK), jnp.int32),
            pltpu.VMEM((N_CHUNKS, CHUNK), jnp.int32),
            pltpu.VMEM((2, CHUNK, BLK), jnp.float32),
            pltpu.VMEM((2, CHUNK, BLK), jnp.float32),
            pltpu.VMEM((BPW,), jnp.float32),
            pltpu.VMEM((BPW,), jnp.float32),
            pltpu.SemaphoreType.DMA,
            pltpu.SemaphoreType.DMA,
        ],
        compiler_params=pltpu.CompilerParams(use_tc_tiling_on_sc=True),
    )(nid, nnid, table_p, bias)


@jax.jit
def _run(node_id, node_neighbor_id, embedding_matrix, bias):
    nid = node_id.astype(jnp.int32)
    nnid = node_neighbor_id.astype(jnp.int32)
    table_p = _pack_table(embedding_matrix.T)
    return _sc_gather_dot(nid, nnid, table_p, bias)


def kernel(node_id, node_neighbor_id, embedding_matrix, bias):
    return _run(node_id, node_neighbor_id, embedding_matrix, bias)


# speedup vs baseline: 1.5183x; 1.0689x over previous
import jax
import jax.numpy as jnp
from jax import lax
from jax.experimental import pallas as pl
from jax.experimental.pallas import tpu as pltpu
from jax.experimental.pallas import tpu_sc as plsc

N_CORES = 2
N_SUBCORES = 16
NW = N_CORES * N_SUBCORES
L = 16

BATCH = 16384
D = 64
BLK = 2 * D
HALF = 50176
NBLK = HALF // 128
BPW = BATCH // NW
CHUNK = 128
N_CHUNKS = BPW // CHUNK
GROUPS = CHUNK // L


PACK_COLS = 6272
PACK_GRID = HALF // PACK_COLS


def _tc_pack_body(t1_ref, t2_ref, out_ref):
    out_ref[:, 0:D] = t1_ref[...].T
    out_ref[:, D:BLK] = t2_ref[...].T


def _pack_table(table_t):
    return pl.pallas_call(
        _tc_pack_body,
        out_shape=jax.ShapeDtypeStruct((HALF, BLK), jnp.float32),
        grid=(PACK_GRID,),
        in_specs=[
            pl.BlockSpec((D, PACK_COLS), lambda j: (0, j)),
            pl.BlockSpec((D, PACK_COLS), lambda j: (0, j + PACK_GRID)),
        ],
        out_specs=pl.BlockSpec((PACK_COLS, BLK), lambda j: (j, 0)),
    )(table_t, table_t)


def _permute(v, idx):
    dn = lax.GatherDimensionNumbers(offset_dims=(), collapsed_slice_dims=(0,),
                                    start_index_map=(0,))
    return lax.gather(v, idx[:, None], dn, (1,),
                      mode=lax.GatherScatterMode.PROMISE_IN_BOUNDS)


def _sc_body(nid_hbm, nnid_hbm, table_hbm, bias_hbm, out_hbm,
             idx_a, idx_b, blk_a, blk_b, rid_b, rows_a, rows_b, bias_v, out_v,
             sem0, sem1):
    wid = lax.axis_index("s") * N_CORES + lax.axis_index("c")
    base = wid * BPW

    pltpu.sync_copy(nid_hbm.at[pl.ds(base, BPW)], idx_a)
    pltpu.sync_copy(nnid_hbm.at[pl.ds(base, BPW)], idx_b)

    for i in range(BPW // L):
        c, w = divmod(i, CHUNK // L)
        va = idx_a[pl.ds(i * L, L)]
        vb = idx_b[pl.ds(i * L, L)]
        blk_a[c, pl.ds(w * L, L)] = jnp.where(va < HALF, va, va - HALF)
        blk_b[c, pl.ds(w * L, L)] = jnp.where(vb < HALF, vb, vb - HALF)
        rid_b[c, pl.ds(w * L, L)] = vb

    sems = [sem0, sem1]
    lanes = lax.iota(jnp.int32, L)

    def fire(c):
        buf = c % 2
        s = sems[buf]
        return [
            pltpu.async_copy(table_hbm.at[blk_a.at[c]], rows_a.at[buf], s),
            pltpu.async_copy(table_hbm.at[blk_b.at[c]], rows_b.at[buf], s),
            pltpu.async_copy(bias_hbm.at[rid_b.at[c]],
                             bias_v.at[pl.ds(c * CHUNK, CHUNK)], s),
        ]

    pending = {0: fire(0)}
    for c in range(N_CHUNKS):
        if c + 1 < N_CHUNKS:
            pending[c + 1] = fire(c + 1)
        for cp in pending.pop(c):
            cp.wait()
        buf = c % 2

        def group_body(g, carry, buf=buf, c=c):
            rbase = g * L
            abs_base = c * CHUNK + rbase
            off_a = jnp.where(idx_a[pl.ds(abs_base, L)] < HALF, 0, D)
            off_b = jnp.where(idx_b[pl.ds(abs_base, L)] < HALF, 0, D)
            vs = []
            for r in range(L):
                oa = off_a[r]
                ob = off_b[r]
                row = rbase + r
                acc = (rows_a[buf, row, pl.ds(oa, L)] *
                       rows_b[buf, row, pl.ds(ob, L)])
                for k in range(1, D // L):
                    acc = acc + (rows_a[buf, row, pl.ds(oa + k * L, L)] *
                                 rows_b[buf, row, pl.ds(ob + k * L, L)])
                vs.append(acc)
            s = L // 2
            while s >= 1:
                mask = (lanes & s) == 0
                pidx = lanes ^ s
                nxt = []
                for i in range(s):
                    a = vs[i] + _permute(vs[i], pidx)
                    b = vs[i + s] + _permute(vs[i + s], pidx)
                    nxt.append(jnp.where(mask, a, b))
                vs = nxt
                s //= 2
            out_v[pl.ds(abs_base, L)] = vs[0] + bias_v[pl.ds(abs_base, L)]
            return carry

        lax.fori_loop(0, GROUPS, group_body, 0)

    pltpu.sync_copy(out_v, out_hbm.at[pl.ds(base, BPW)])


def _sc_gather_dot(nid, nnid, table_p, bias):
    mesh = plsc.VectorSubcoreMesh(core_axis_name="c", subcore_axis_name="s")
    return pl.kernel(
        _sc_body,
        out_type=jax.ShapeDtypeStruct((BATCH,), jnp.float32),
        mesh=mesh,
        scratch_types=[
            pltpu.VMEM((BPW,), jnp.int32),
            pltpu.VMEM((BPW,), jnp.int32),
            pltpu.VMEM((N_CHUNKS, CHUNK), jnp.int32),
            pltpu.VMEM((N_CHUNKS, CHUNK), jnp.int32),
            pltpu.VMEM((N_CHUNKS, CHUNK), jnp.int32),
            pltpu.VMEM((2, CHUNK, BLK), jnp.float32),
            pltpu.VMEM((2, CHUNK, BLK), jnp.float32),
            pltpu.VMEM((BPW,), jnp.float32),
            pltpu.VMEM((BPW,), jnp.float32),
            pltpu.SemaphoreType.DMA,
            pltpu.SemaphoreType.DMA,
        ],
        compiler_params=pltpu.CompilerParams(use_tc_tiling_on_sc=True),
    )(nid, nnid, table_p, bias)


@jax.jit
def _run(node_id, node_neighbor_id, embedding_matrix, bias):
    nid = node_id.astype(jnp.int32)
    nnid = node_neighbor_id.astype(jnp.int32)
    table_p = _pack_table(embedding_matrix.T)
    return _sc_gather_dot(nid, nnid, table_p, bias)


def kernel(node_id, node_neighbor_id, embedding_matrix, bias):
    return _run(node_id, node_neighbor_id, embedding_matrix, bias)


# speedup vs baseline: 1.5273x; 1.0059x over previous
import jax
import jax.numpy as jnp
from jax import lax
from jax.experimental import pallas as pl
from jax.experimental.pallas import tpu as pltpu
from jax.experimental.pallas import tpu_sc as plsc

N_CORES = 2
N_SUBCORES = 16
NW = N_CORES * N_SUBCORES
L = 16

BATCH = 16384
D = 64
BLK = 2 * D
HALF = 50176
NBLK = HALF // 128
BPW = BATCH // NW
CHUNK = 128
N_CHUNKS = BPW // CHUNK
GROUPS = CHUNK // L


PACK_COLS = 12544
PACK_GRID = HALF // PACK_COLS


def _tc_pack_body(t1_ref, t2_ref, out_ref):
    out_ref[:, 0:D] = t1_ref[...].T
    out_ref[:, D:BLK] = t2_ref[...].T


def _pack_table(table_t):
    return pl.pallas_call(
        _tc_pack_body,
        out_shape=jax.ShapeDtypeStruct((HALF, BLK), jnp.float32),
        grid=(PACK_GRID,),
        in_specs=[
            pl.BlockSpec((D, PACK_COLS), lambda j: (0, j)),
            pl.BlockSpec((D, PACK_COLS), lambda j: (0, j + PACK_GRID)),
        ],
        out_specs=pl.BlockSpec((PACK_COLS, BLK), lambda j: (j, 0)),
    )(table_t, table_t)


def _permute(v, idx):
    dn = lax.GatherDimensionNumbers(offset_dims=(), collapsed_slice_dims=(0,),
                                    start_index_map=(0,))
    return lax.gather(v, idx[:, None], dn, (1,),
                      mode=lax.GatherScatterMode.PROMISE_IN_BOUNDS)


def _sc_body(nid_hbm, nnid_hbm, table_hbm, bias_hbm, out_hbm,
             idx_a, idx_b, blk_a, blk_b, rid_b, rows_a, rows_b, bias_v, out_v,
             sem0, sem1):
    wid = lax.axis_index("s") * N_CORES + lax.axis_index("c")
    base = wid * BPW

    pltpu.sync_copy(nid_hbm.at[pl.ds(base, BPW)], idx_a)
    pltpu.sync_copy(nnid_hbm.at[pl.ds(base, BPW)], idx_b)

    for i in range(BPW // L):
        c, w = divmod(i, CHUNK // L)
        va = idx_a[pl.ds(i * L, L)]
        vb = idx_b[pl.ds(i * L, L)]
        blk_a[c, pl.ds(w * L, L)] = jnp.where(va < HALF, va, va - HALF)
        blk_b[c, pl.ds(w * L, L)] = jnp.where(vb < HALF, vb, vb - HALF)
        rid_b[c, pl.ds(w * L, L)] = vb

    sems = [sem0, sem1]
    lanes = lax.iota(jnp.int32, L)

    def fire(c):
        buf = c % 2
        s = sems[buf]
        return [
            pltpu.async_copy(table_hbm.at[blk_a.at[c]], rows_a.at[buf], s),
            pltpu.async_copy(table_hbm.at[blk_b.at[c]], rows_b.at[buf], s),
            pltpu.async_copy(bias_hbm.at[rid_b.at[c]],
                             bias_v.at[pl.ds(c * CHUNK, CHUNK)], s),
        ]

    pending = {0: fire(0)}
    for c in range(N_CHUNKS):
        if c + 1 < N_CHUNKS:
            pending[c + 1] = fire(c + 1)
        for cp in pending.pop(c):
            cp.wait()
        buf = c % 2

        def group_body(g, carry, buf=buf, c=c):
            rbase = g * L
            abs_base = c * CHUNK + rbase
            off_a = jnp.where(idx_a[pl.ds(abs_base, L)] < HALF, 0, D)
            off_b = jnp.where(idx_b[pl.ds(abs_base, L)] < HALF, 0, D)
            vs = []
            for r in range(L):
                oa = off_a[r]
                ob = off_b[r]
                row = rbase + r
                acc = (rows_a[buf, row, pl.ds(oa, L)] *
                       rows_b[buf, row, pl.ds(ob, L)])
                for k in range(1, D // L):
                    acc = acc + (rows_a[buf, row, pl.ds(oa + k * L, L)] *
                                 rows_b[buf, row, pl.ds(ob + k * L, L)])
                vs.append(acc)
            s = L // 2
            while s >= 1:
                mask = (lanes & s) == 0
                pidx = lanes ^ s
                nxt = []
                for i in range(s):
                    a = vs[i] + _permute(vs[i], pidx)
                    b = vs[i + s] + _permute(vs[i + s], pidx)
                    nxt.append(jnp.where(mask, a, b))
                vs = nxt
                s //= 2
            out_v[pl.ds(abs_base, L)] = vs[0] + bias_v[pl.ds(abs_base, L)]
            return carry

        lax.fori_loop(0, GROUPS, group_body, 0)

    pltpu.sync_copy(out_v, out_hbm.at[pl.ds(base, BPW)])


def _sc_gather_dot(nid, nnid, table_p, bias):
    mesh = plsc.VectorSubcoreMesh(core_axis_name="c", subcore_axis_name="s")
    return pl.kernel(
        _sc_body,
        out_type=jax.ShapeDtypeStruct((BATCH,), jnp.float32),
        mesh=mesh,
        scratch_types=[
            pltpu.VMEM((BPW,), jnp.int32),
            pltpu.VMEM((BPW,), jnp.int32),
            pltpu.VMEM((N_CHUNKS, CHUNK), jnp.int32),
            pltpu.VMEM((N_CHUNKS, CHUNK), jnp.int32),
            pltpu.VMEM((N_CHUNKS, CHUNK), jnp.int32),
            pltpu.VMEM((2, CHUNK, BLK), jnp.float32),
            pltpu.VMEM((2, CHUNK, BLK), jnp.float32),
            pltpu.VMEM((BPW,), jnp.float32),
            pltpu.VMEM((BPW,), jnp.float32),
            pltpu.SemaphoreType.DMA,
            pltpu.SemaphoreType.DMA,
        ],
        compiler_params=pltpu.CompilerParams(use_tc_tiling_on_sc=True),
    )(nid, nnid, table_p, bias)


@jax.jit
def _run(node_id, node_neighbor_id, embedding_matrix, bias):
    nid = node_id.astype(jnp.int32)
    nnid = node_neighbor_id.astype(jnp.int32)
    table_p = _pack_table(embedding_matrix.T)
    return _sc_gather_dot(nid, nnid, table_p, bias)


def kernel(node_id, node_neighbor_id, embedding_matrix, bias):
    return _run(node_id, node_neighbor_id, embedding_matrix, bias)
